# baseline (speedup 1.0000x reference)
import functools

import jax
import jax.numpy as jnp
import numpy as np
from jax import lax
from jax.experimental import pallas as pl
from jax.experimental.pallas import tpu as pltpu
from jax.experimental.pallas import tpu_sc as plsc

CHUNK = 8


def _pe_table(seq_len, d_model):
    pos = np.arange(seq_len, dtype=np.float32)[:, None]
    i = np.arange(0, d_model, 2, dtype=np.float32)
    angle = pos / np.power(10000.0, i / d_model)
    pe = np.zeros((seq_len, d_model), dtype=np.float32)
    pe[:, 0::2] = np.sin(angle)
    pe[:, 1::2] = np.cos(angle)
    return pe


@functools.cache
def _build(batch, seq, vocab, d_model):
    try:
        info = plsc.get_sparse_core_info()
        num_cores, num_subcores = info.num_cores, info.num_subcores
    except Exception:
        num_cores, num_subcores = 2, 16
    nw = num_cores * num_subcores
    s_per_w = seq // nw
    chunk = min(CHUNK, s_per_w)
    n_steps = s_per_w // chunk
    n_vec = d_model // 16
    rows = batch * chunk
    mesh = plsc.VectorSubcoreMesh(core_axis_name="c", subcore_axis_name="s")

    nbuf = 3
    scratch = (
        [pltpu.VMEM((batch * s_per_w,), jnp.int32)]
        + [pltpu.VMEM((rows, d_model), jnp.float32) for _ in range(nbuf)]
        + [pltpu.VMEM((chunk, d_model), jnp.float32) for _ in range(nbuf)]
        + [pltpu.SemaphoreType.DMA for _ in range(2 * nbuf)]
    )

    @functools.partial(
        pl.kernel,
        mesh=mesh,
        out_type=jax.ShapeDtypeStruct((batch, seq, d_model), jnp.float32),
        scratch_types=scratch,
    )
    def emb(table_hbm, x_hbm, pe_hbm, out_hbm, idx2_v, *bufs):
        tok_v = [bufs[pp] for pp in range(nbuf)]
        pe_v = [bufs[nbuf + pp] for pp in range(nbuf)]
        gsem = [bufs[2 * nbuf + pp] for pp in range(nbuf)]
        osem = [bufs[3 * nbuf + pp] for pp in range(nbuf)]

        wid = lax.axis_index("s") * num_cores + lax.axis_index("c")
        s0 = wid * s_per_w
        pltpu.sync_copy(x_hbm.at[wid], idx2_v)

        gathers = {}
        outs = {}

        def issue_gathers(ch):
            pp = ch % nbuf
            gathers[ch] = [
                pltpu.async_copy(
                    table_hbm.at[idx2_v.at[pl.ds(ch * rows, rows)]],
                    tok_v[pp],
                    gsem[pp],
                ),
                pltpu.async_copy(
                    pe_hbm.at[pl.ds(s0 + ch * chunk, chunk), :],
                    pe_v[pp],
                    gsem[pp],
                ),
            ]

        for ch in range(min(nbuf - 1, n_steps)):
            issue_gathers(ch)
        for ch in range(n_steps):
            pp = ch % nbuf
            for d in gathers.pop(ch):
                d.wait()

            pe_b = pe_v[pp]
            tok_b = tok_v[pp]

            @plsc.parallel_loop(0, chunk * n_vec, 1, unroll=4)
            def add_pe(i):
                r = i // n_vec
                off = (i % n_vec) * 16
                pvec = pe_b[r, pl.ds(off, 16)]
                for b in range(batch):
                    tok_b[b * chunk + r, pl.ds(off, 16)] = (
                        tok_b[b * chunk + r, pl.ds(off, 16)] + pvec
                    )

            outs[ch] = [
                pltpu.async_copy(
                    tok_v[pp].at[pl.ds(b * chunk, chunk), :],
                    out_hbm.at[b, pl.ds(s0 + ch * chunk, chunk), :],
                    osem[pp],
                )
                for b in range(batch)
            ]
            nxt = ch + nbuf - 1
            if nxt < n_steps:
                if nxt - nbuf >= 0:
                    for d in outs.pop(nxt - nbuf):
                        d.wait()
                issue_gathers(nxt)
        for ch in sorted(outs):
            for d in outs[ch]:
                d.wait()

    def run(x_i32, table, pe):
        x2 = (
            x_i32.reshape(batch, nw, n_steps, chunk)
            .transpose(1, 2, 0, 3)
            .reshape(nw, batch * s_per_w)
        )
        return emb(table, x2, pe)

    return run


def kernel(x, tok_table):
    batch, seq = x.shape
    vocab, d_model = tok_table.shape
    pe = jnp.asarray(_pe_table(seq, d_model))
    run = _build(batch, seq, vocab, d_model)
    return run(x.astype(jnp.int32), tok_table, pe)
